# baseline (speedup 1.0000x reference)
import functools

import jax
import jax.numpy as jnp
from jax import lax
from jax.experimental import pallas as pl
from jax.experimental.pallas import tpu as pltpu
from jax.experimental.pallas import tpu_sc as plsc

N = 10000
E = 320000
D = 128
NC, NS = 2, 16
NW = NC * NS
CHUNK = 128
RPT = 80
EC = NW * RPT
E_PAD = EC * CHUNK
NPAD = 10240
ROWS_PT = NPAD // NS
DD = 16
IB = 16

_MESH = plsc.VectorSubcoreMesh(
    core_axis_name="c", subcore_axis_name="s", num_cores=NC, num_subcores=NS)


def _zero_buf(buf, nrows, width):
    z = jnp.zeros((16,), jnp.float32)

    def body(r, _):
        for j in range(width // 16):
            buf[r, pl.ds(j * 16, 16)] = z
        return 0

    lax.fori_loop(0, nrows, body, 0)


@functools.partial(
    pl.kernel,
    out_type=jax.ShapeDtypeStruct((NC, NPAD, DD), jnp.float32),
    mesh=_MESH,
    scratch_types=[
        pltpu.VMEM_SHARED((NPAD, DD), jnp.float32),
        pltpu.VMEM((RPT, CHUNK), jnp.int32),
        pltpu.VMEM((CHUNK, DD), jnp.float32),
        pltpu.VMEM((CHUNK, DD), jnp.float32),
    ],
)
def _sc_deg(dst_h, out_h, acc_sh, didx, obuf, zbuf):
    cid = lax.axis_index("c")
    sid = lax.axis_index("s")
    wid = cid * NS + sid

    def fill(r, _):
        obuf[r, pl.ds(0, 16)] = jnp.full((16,), 0.0625, jnp.float32)
        zbuf[r, pl.ds(0, 16)] = jnp.zeros((16,), jnp.float32)
        return 0
    lax.fori_loop(0, CHUNK, fill, 0)

    for t in range(ROWS_PT // CHUNK):
        pltpu.sync_copy(zbuf, acc_sh.at[pl.ds(sid * ROWS_PT + t * CHUNK, CHUNK)])
    plsc.subcore_barrier()

    pltpu.sync_copy(dst_h.at[pl.ds(wid * RPT, RPT)], didx)

    def body(j, _):
        pltpu.sync_copy(obuf, acc_sh.at[didx.at[j]], add=True)
        return 0
    lax.fori_loop(0, RPT, body, 0)
    plsc.subcore_barrier()

    pltpu.sync_copy(acc_sh.at[pl.ds(sid * ROWS_PT, ROWS_PT)],
                    out_h.at[cid, pl.ds(sid * ROWS_PT, ROWS_PT)])


@functools.partial(
    pl.kernel,
    out_type=jax.ShapeDtypeStruct((NC, NPAD, D), jnp.float32),
    mesh=_MESH,
    scratch_types=[
        pltpu.VMEM_SHARED((NPAD, D), jnp.float32),
        pltpu.VMEM((IB, CHUNK), jnp.int32),
        pltpu.VMEM((IB, CHUNK), jnp.int32),
        pltpu.VMEM((2, CHUNK, D), jnp.float32),
        pltpu.SemaphoreType.DMA,
        pltpu.SemaphoreType.DMA,
        pltpu.SemaphoreType.DMA,
        pltpu.SemaphoreType.DMA,
    ],
)
def _sc_agg(table_h, src_h, dst_h, out_h, acc_sh, sidx, didx, rows,
            sem0, sem1, sem2, sem3):
    cid = lax.axis_index("c")
    sid = lax.axis_index("s")
    wid = cid * NS + sid
    sems = ((sem0, sem1), (sem2, sem3))
    HH = CHUNK // 2

    def gissue(j, b):
        for hf in range(2):
            pltpu.async_copy(table_h.at[sidx.at[j, pl.ds(hf * HH, HH)]],
                             rows.at[b, pl.ds(hf * HH, HH)], sems[b][hf])

    def gwait(j, b):
        for hf in range(2):
            pltpu.make_async_copy(table_h.at[sidx.at[j, pl.ds(hf * HH, HH)]],
                                  rows.at[b, pl.ds(hf * HH, HH)],
                                  sems[b][hf]).wait()

    _zero_buf(rows.at[0], CHUNK, D)
    for t in range(ROWS_PT // CHUNK):
        pltpu.sync_copy(rows.at[0], acc_sh.at[pl.ds(sid * ROWS_PT + t * CHUNK, CHUNK)])
    plsc.subcore_barrier()

    def block(t, _):
        base = wid * RPT + t * IB
        pltpu.sync_copy(src_h.at[pl.ds(base, IB)], sidx)
        pltpu.sync_copy(dst_h.at[pl.ds(base, IB)], didx)
        gissue(0, 0)
        gissue(1, 1)

        def body(i, _):
            for b in range(2):
                j = 2 * i + b
                gwait(j, b)
                pltpu.sync_copy(rows.at[b], acc_sh.at[didx.at[j]], add=True)
                gissue(j + 2, b)
            return 0

        lax.fori_loop(0, IB // 2 - 1, body, 0)
        gwait(IB - 2, 0)
        pltpu.sync_copy(rows.at[0], acc_sh.at[didx.at[IB - 2]], add=True)
        gwait(IB - 1, 1)
        pltpu.sync_copy(rows.at[1], acc_sh.at[didx.at[IB - 1]], add=True)
        return 0

    lax.fori_loop(0, RPT // IB, block, 0)
    plsc.subcore_barrier()

    pltpu.sync_copy(acc_sh.at[pl.ds(sid * ROWS_PT, ROWS_PT)],
                    out_h.at[cid, pl.ds(sid * ROWS_PT, ROWS_PT)])


def _dinv_col(deg_ref):
    s = jnp.sum(deg_ref[0] + deg_ref[1], axis=1)
    return lax.rsqrt(1.0 + s)[:, None]


def _tc_in_body(x_ref, deg_ref, w_ref, o_ref):
    h = jnp.dot(x_ref[...], w_ref[...], preferred_element_type=jnp.float32)
    o_ref[...] = h * _dinv_col(deg_ref)


def _tc_mid_body(acc_ref, hs_ref, deg_ref, b_ref, w_ref, o_ref):
    dinv = _dinv_col(deg_ref)
    z = (acc_ref[0] + acc_ref[1] + hs_ref[...]) * dinv + b_ref[...]
    z = jnp.maximum(z, 0.0)
    o_ref[...] = jnp.dot(z, w_ref[...], preferred_element_type=jnp.float32) * dinv


def _tc_out_body(acc_ref, hs_ref, deg_ref, b_ref, o_ref):
    dinv = _dinv_col(deg_ref)
    o_ref[...] = (acc_ref[0] + acc_ref[1] + hs_ref[...]) * dinv + b_ref[...]


_R = 1000
_fspec = pl.BlockSpec((_R, D), lambda i: (i, 0))
_aspec = pl.BlockSpec((NC, _R, D), lambda i: (0, i, 0))
_dspec = pl.BlockSpec((NC, _R, DD), lambda i: (0, i, 0))
_wspec = pl.BlockSpec((D, D), lambda i: (0, 0))
_bspec = pl.BlockSpec((1, D), lambda i: (0, 0))
_fout = jax.ShapeDtypeStruct((N, D), jnp.float32)


def _tc_in(x, degp, W1):
    return pl.pallas_call(
        _tc_in_body, grid=(N // _R,),
        in_specs=[_fspec, _dspec, _wspec], out_specs=_fspec, out_shape=_fout,
    )(x, degp, W1)


def _tc_mid(acc, hs, degp, b, W2):
    return pl.pallas_call(
        _tc_mid_body, grid=(N // _R,),
        in_specs=[_aspec, _fspec, _dspec, _bspec, _wspec],
        out_specs=_fspec, out_shape=_fout,
    )(acc, hs, degp, b, W2)


def _tc_out(acc, hs, degp, b):
    return pl.pallas_call(
        _tc_out_body, grid=(N // _R,),
        in_specs=[_aspec, _fspec, _dspec, _bspec],
        out_specs=_fspec, out_shape=_fout,
    )(acc, hs, degp, b)


def kernel(x, edge_index, W1, b1, W2, b2):
    src = edge_index[0].astype(jnp.int32)
    dst = edge_index[1].astype(jnp.int32)
    pad = E_PAD - E
    ar = jnp.arange(pad, dtype=jnp.int32)
    srcc = jnp.concatenate([src, ar % N]).reshape(EC, CHUNK)
    dstc = jnp.concatenate([dst, jnp.full((pad,), N, jnp.int32)]).reshape(EC, CHUNK)

    degp = _sc_deg(dstc)
    h1s = _tc_in(x, degp, W1)
    acc1 = _sc_agg(h1s, srcc, dstc)
    h2s = _tc_mid(acc1, h1s, degp, b1.reshape(1, D), W2)
    acc2 = _sc_agg(h2s, srcc, dstc)
    return _tc_out(acc2, h2s, degp, b2.reshape(1, D))


# speedup vs baseline: 1.0332x; 1.0332x over previous
import functools

import jax
import jax.numpy as jnp
from jax import lax
from jax.experimental import pallas as pl
from jax.experimental.pallas import tpu as pltpu
from jax.experimental.pallas import tpu_sc as plsc

N = 10000
E = 320000
D = 128
NC, NS = 2, 16
NW = NC * NS
CHUNK = 128
RPT = 80
EC = NW * RPT
E_PAD = EC * CHUNK
NPAD = 10240
ROWS_PT = NPAD // NS
DD = 16
IB = 16

_MESH = plsc.VectorSubcoreMesh(
    core_axis_name="c", subcore_axis_name="s", num_cores=NC, num_subcores=NS)


def _zero_buf(buf, nrows, width):
    z = jnp.zeros((16,), jnp.float32)

    def body(r, _):
        for j in range(width // 16):
            buf[r, pl.ds(j * 16, 16)] = z
        return 0

    lax.fori_loop(0, nrows, body, 0)


@functools.partial(
    pl.kernel,
    out_type=jax.ShapeDtypeStruct((NC, NPAD, DD), jnp.float32),
    mesh=_MESH,
    scratch_types=[
        pltpu.VMEM_SHARED((NPAD, DD), jnp.float32),
        pltpu.VMEM((RPT, CHUNK), jnp.int32),
        pltpu.VMEM((CHUNK, DD), jnp.float32),
        pltpu.VMEM((CHUNK, DD), jnp.float32),
    ],
)
def _sc_deg(dst_h, out_h, acc_sh, didx, obuf, zbuf):
    cid = lax.axis_index("c")
    sid = lax.axis_index("s")
    wid = cid * NS + sid

    def fill(r, _):
        obuf[r, pl.ds(0, 16)] = jnp.full((16,), 0.0625, jnp.float32)
        zbuf[r, pl.ds(0, 16)] = jnp.zeros((16,), jnp.float32)
        return 0
    lax.fori_loop(0, CHUNK, fill, 0)

    for t in range(ROWS_PT // CHUNK):
        pltpu.sync_copy(zbuf, acc_sh.at[pl.ds(sid * ROWS_PT + t * CHUNK, CHUNK)])
    plsc.subcore_barrier()

    pltpu.sync_copy(dst_h.at[pl.ds(wid * RPT, RPT)], didx)

    def body(j, _):
        pltpu.sync_copy(obuf, acc_sh.at[didx.at[j]], add=True)
        return 0
    lax.fori_loop(0, RPT, body, 0)
    plsc.subcore_barrier()

    pltpu.sync_copy(acc_sh.at[pl.ds(sid * ROWS_PT, ROWS_PT)],
                    out_h.at[cid, pl.ds(sid * ROWS_PT, ROWS_PT)])


@functools.partial(
    pl.kernel,
    out_type=jax.ShapeDtypeStruct((NC, NPAD, D), jnp.float32),
    mesh=_MESH,
    scratch_types=[
        pltpu.VMEM_SHARED((NPAD, D), jnp.float32),
        pltpu.VMEM((IB, CHUNK), jnp.int32),
        pltpu.VMEM((IB, CHUNK), jnp.int32),
        pltpu.VMEM((2, CHUNK, D), jnp.float32),
        pltpu.SemaphoreType.DMA,
        pltpu.SemaphoreType.DMA,
    ],
)
def _sc_agg(table_h, src_h, dst_h, out_h, acc_sh, sidx, didx, rows, sem0, sem1):
    cid = lax.axis_index("c")
    sid = lax.axis_index("s")
    wid = cid * NS + sid
    sems = (sem0, sem1)

    _zero_buf(rows.at[0], CHUNK, D)
    for t in range(ROWS_PT // CHUNK):
        pltpu.sync_copy(rows.at[0], acc_sh.at[pl.ds(sid * ROWS_PT + t * CHUNK, CHUNK)])
    plsc.subcore_barrier()

    def block(t, _):
        base = wid * RPT + t * IB
        pltpu.sync_copy(src_h.at[pl.ds(base, IB)], sidx)
        pltpu.sync_copy(dst_h.at[pl.ds(base, IB)], didx)
        pltpu.async_copy(table_h.at[sidx.at[0]], rows.at[0], sem0)

        def body(i, _):
            for b in range(2):
                j = 2 * i + b
                pltpu.async_copy(table_h.at[sidx.at[j + 1]], rows.at[1 - b],
                                 sems[1 - b])
                pltpu.make_async_copy(table_h.at[sidx.at[j]], rows.at[b],
                                      sems[b]).wait()
                pltpu.sync_copy(rows.at[b], acc_sh.at[didx.at[j]], add=True)
            return 0

        lax.fori_loop(0, IB // 2 - 1, body, 0)
        pltpu.async_copy(table_h.at[sidx.at[IB - 1]], rows.at[1], sem1)
        pltpu.make_async_copy(table_h.at[sidx.at[IB - 2]], rows.at[0], sem0).wait()
        pltpu.sync_copy(rows.at[0], acc_sh.at[didx.at[IB - 2]], add=True)
        pltpu.make_async_copy(table_h.at[sidx.at[IB - 1]], rows.at[1], sem1).wait()
        pltpu.sync_copy(rows.at[1], acc_sh.at[didx.at[IB - 1]], add=True)
        return 0

    lax.fori_loop(0, RPT // IB, block, 0)
    plsc.subcore_barrier()

    pltpu.sync_copy(acc_sh.at[pl.ds(sid * ROWS_PT, ROWS_PT)],
                    out_h.at[cid, pl.ds(sid * ROWS_PT, ROWS_PT)])


def _dinv_col(deg_ref):
    s = jnp.sum(deg_ref[0] + deg_ref[1], axis=1)
    return lax.rsqrt(1.0 + s)[:, None]


def _tc_in_body(x_ref, deg_ref, w_ref, o_ref):
    h = jnp.dot(x_ref[...], w_ref[...], preferred_element_type=jnp.float32)
    o_ref[...] = h * _dinv_col(deg_ref)


def _tc_mid_body(acc_ref, hs_ref, deg_ref, b_ref, w_ref, o_ref):
    dinv = _dinv_col(deg_ref)
    z = (acc_ref[0] + acc_ref[1] + hs_ref[...]) * dinv + b_ref[...]
    z = jnp.maximum(z, 0.0)
    o_ref[...] = jnp.dot(z, w_ref[...], preferred_element_type=jnp.float32) * dinv


def _tc_out_body(acc_ref, hs_ref, deg_ref, b_ref, o_ref):
    dinv = _dinv_col(deg_ref)
    o_ref[...] = (acc_ref[0] + acc_ref[1] + hs_ref[...]) * dinv + b_ref[...]


_R = 2000
_fspec = pl.BlockSpec((_R, D), lambda i: (i, 0))
_aspec = pl.BlockSpec((NC, _R, D), lambda i: (0, i, 0))
_dspec = pl.BlockSpec((NC, _R, DD), lambda i: (0, i, 0))
_wspec = pl.BlockSpec((D, D), lambda i: (0, 0))
_bspec = pl.BlockSpec((1, D), lambda i: (0, 0))
_fout = jax.ShapeDtypeStruct((N, D), jnp.float32)


def _tc_in(x, degp, W1):
    return pl.pallas_call(
        _tc_in_body, grid=(N // _R,),
        in_specs=[_fspec, _dspec, _wspec], out_specs=_fspec, out_shape=_fout,
    )(x, degp, W1)


def _tc_mid(acc, hs, degp, b, W2):
    return pl.pallas_call(
        _tc_mid_body, grid=(N // _R,),
        in_specs=[_aspec, _fspec, _dspec, _bspec, _wspec],
        out_specs=_fspec, out_shape=_fout,
    )(acc, hs, degp, b, W2)


def _tc_out(acc, hs, degp, b):
    return pl.pallas_call(
        _tc_out_body, grid=(N // _R,),
        in_specs=[_aspec, _fspec, _dspec, _bspec],
        out_specs=_fspec, out_shape=_fout,
    )(acc, hs, degp, b)


def kernel(x, edge_index, W1, b1, W2, b2):
    src = edge_index[0].astype(jnp.int32)
    dst = edge_index[1].astype(jnp.int32)
    pad = E_PAD - E
    ar = jnp.arange(pad, dtype=jnp.int32)
    srcc = jnp.concatenate([src, ar % N]).reshape(EC, CHUNK)
    dstc = jnp.concatenate([dst, jnp.full((pad,), N, jnp.int32)]).reshape(EC, CHUNK)

    degp = _sc_deg(dstc)
    h1s = _tc_in(x, degp, W1)
    acc1 = _sc_agg(h1s, srcc, dstc)
    h2s = _tc_mid(acc1, h1s, degp, b1.reshape(1, D), W2)
    acc2 = _sc_agg(h2s, srcc, dstc)
    return _tc_out(acc2, h2s, degp, b2.reshape(1, D))
